# baseline (speedup 1.0000x reference)
import functools

import jax
import jax.numpy as jnp
import numpy as np
from jax import lax
from jax.experimental import pallas as pl
from jax.experimental.pallas import tpu as pltpu
from jax.experimental.pallas import tpu_sc as plsc

_NUM_CORES = 2
_NUM_SUBCORES = 16
_NW = _NUM_CORES * _NUM_SUBCORES
_NCHUNK = 4
_LANES = 16
_ROW_UNROLL = 4


def _pack_permutation(d):
    perm = np.arange(d).reshape(-1, 2, _LANES).transpose(0, 2, 1).reshape(-1)
    return perm


def _gather_rows_sc_bf16(table, idx):
    B = idx.shape[0]
    D = table.shape[1]
    b_per_w = B // _NW
    rows_per_chunk = b_per_w // _NCHUNK
    groups_per_row = D // (2 * _LANES)
    idx3 = idx.reshape(_NW, _NCHUNK, rows_per_chunk)

    mesh = plsc.VectorSubcoreMesh(
        core_axis_name="c", subcore_axis_name="s",
        num_cores=_NUM_CORES, num_subcores=_NUM_SUBCORES)

    dw = D // 2

    @functools.partial(
        pl.kernel,
        mesh=mesh,
        compiler_params=pltpu.CompilerParams(needs_layout_passes=False),
        out_type=jax.ShapeDtypeStruct((B * dw,), jnp.uint32),
        scratch_types=[
            pltpu.VMEM((_NCHUNK, rows_per_chunk), jnp.int32),
            pltpu.VMEM((b_per_w, D), jnp.float32),
            pltpu.VMEM((b_per_w * dw,), jnp.uint32),
            pltpu.SemaphoreType.DMA,
            pltpu.SemaphoreType.DMA,
        ],
    )
    def gather_kernel(table_hbm, idx_hbm, out_hbm, idx_v, rows_v, bf_v,
                      gsem, wsem):
        wid = lax.axis_index("s") * _NUM_CORES + lax.axis_index("c")
        base = wid * b_per_w
        pltpu.sync_copy(idx_hbm.at[wid], idx_v)
        gathers = []
        for c in range(_NCHUNK):
            gathers.append(pltpu.async_copy(
                table_hbm.at[idx_v.at[c]],
                rows_v.at[pl.ds(c * rows_per_chunk, rows_per_chunk)],
                gsem))
        writes = []
        chunk_words = rows_per_chunk * dw
        hi_mask = jnp.uint32(0xFFFF0000)
        for c in range(_NCHUNK):
            gathers[c].wait()

            def convert_rows(r, carry, c=c):
                for j in range(_ROW_UNROLL):
                    row = c * rows_per_chunk + r * _ROW_UNROLL + j
                    for g in range(groups_per_row):
                        a = rows_v[row, pl.ds(g * 2 * _LANES, _LANES)]
                        b = rows_v[row, pl.ds(g * 2 * _LANES + _LANES,
                                              _LANES)]
                        au = plsc.bitcast(a, jnp.uint32)
                        bu = plsc.bitcast(b, jnp.uint32)
                        packed = (au >> 16) | (bu & hi_mask)
                        bf_v[pl.ds(row * dw + g * _LANES, _LANES)] = packed
                return carry

            lax.fori_loop(0, rows_per_chunk // _ROW_UNROLL, convert_rows, 0)
            writes.append(pltpu.async_copy(
                bf_v.at[pl.ds(c * chunk_words, chunk_words)],
                out_hbm.at[pl.ds(base * dw + c * chunk_words, chunk_words)],
                wsem))
        for w in writes:
            w.wait()

    packed_flat = gather_kernel(table, idx3)
    return lax.bitcast_convert_type(packed_flat, jnp.bfloat16).reshape(B, D)


def _mlp_tc(x, w1, b1, w2, b2, blk):
    B, D = x.shape
    H, O = w1.shape[0], w2.shape[0]
    contract_t = (((1,), (1,)), ((), ()))

    def body(x_ref, w1_ref, b1_ref, w2_ref, b2_ref, out_ref):
        xf = x_ref[...].astype(jnp.float32)
        h = lax.dot_general(xf, w1_ref[...], contract_t,
                            preferred_element_type=jnp.float32)
        h = jnp.maximum(h + b1_ref[...], 0.0)
        out_ref[...] = lax.dot_general(h, w2_ref[...], contract_t,
                                       preferred_element_type=jnp.float32
                                       ) + b2_ref[...]

    return pl.pallas_call(
        body,
        grid=(B // blk,),
        in_specs=[
            pl.BlockSpec((blk, D), lambda i: (i, 0)),
            pl.BlockSpec((H, D), lambda i: (0, 0)),
            pl.BlockSpec((1, H), lambda i: (0, 0)),
            pl.BlockSpec((O, H), lambda i: (0, 0)),
            pl.BlockSpec((1, O), lambda i: (0, 0)),
        ],
        out_specs=pl.BlockSpec((blk, O), lambda i: (i, 0)),
        out_shape=jax.ShapeDtypeStruct((B, O), jnp.float32),
    )(x, w1, b1.reshape(1, H), w2, b2.reshape(1, O))


def kernel(user_idx, table, W1, b1, W2, b2):
    x = _gather_rows_sc_bf16(table, user_idx.astype(jnp.int32))
    w1p = W1[:, _pack_permutation(W1.shape[1])]
    return _mlp_tc(x, w1p, b1, W2, b2, blk=8192)


# speedup vs baseline: 2.4819x; 2.4819x over previous
import functools

import jax
import jax.numpy as jnp
from jax import lax
from jax.experimental import pallas as pl
from jax.experimental.pallas import tpu as pltpu
from jax.experimental.pallas import tpu_sc as plsc

_NUM_CORES = 2
_NUM_SUBCORES = 16
_NW = _NUM_CORES * _NUM_SUBCORES
_NCHUNK = 1


def _gather_rows_sc(table, idx):
    B = idx.shape[0]
    D = table.shape[1]
    b_per_w = B // _NW
    rows_per_chunk = b_per_w // _NCHUNK
    idx3 = idx.reshape(_NW, _NCHUNK, rows_per_chunk)

    mesh = plsc.VectorSubcoreMesh(
        core_axis_name="c", subcore_axis_name="s",
        num_cores=_NUM_CORES, num_subcores=_NUM_SUBCORES)

    @functools.partial(
        pl.kernel,
        mesh=mesh,
        out_type=jax.ShapeDtypeStruct((B, D), jnp.float32),
        scratch_types=[
            pltpu.VMEM((_NCHUNK, rows_per_chunk), jnp.int32),
            pltpu.VMEM((b_per_w, D), jnp.float32),
            pltpu.SemaphoreType.DMA,
            pltpu.SemaphoreType.DMA,
        ],
    )
    def gather_kernel(table_hbm, idx_hbm, out_hbm, idx_v, rows_v, gsem, wsem):
        wid = lax.axis_index("s") * _NUM_CORES + lax.axis_index("c")
        base = wid * b_per_w
        pltpu.sync_copy(idx_hbm.at[wid], idx_v)
        gathers = []
        for c in range(_NCHUNK):
            gathers.append(pltpu.async_copy(
                table_hbm.at[idx_v.at[c]],
                rows_v.at[pl.ds(c * rows_per_chunk, rows_per_chunk)],
                gsem))
        writes = []
        for c in range(_NCHUNK):
            gathers[c].wait()
            writes.append(pltpu.async_copy(
                rows_v.at[pl.ds(c * rows_per_chunk, rows_per_chunk)],
                out_hbm.at[pl.ds(base + c * rows_per_chunk, rows_per_chunk)],
                wsem))
        for w in writes:
            w.wait()

    return gather_kernel(table, idx3)


def _mlp_tc(x, w1, b1, w2, b2, blk):
    B, D = x.shape
    H, O = w1.shape[0], w2.shape[0]
    contract_t = (((1,), (1,)), ((), ()))

    def body(x_ref, w1_ref, b1_ref, w2_ref, b2_ref, out_ref):
        h = lax.dot_general(x_ref[...], w1_ref[...], contract_t,
                            preferred_element_type=jnp.float32)
        h = jnp.maximum(h + b1_ref[...], 0.0)
        out_ref[...] = lax.dot_general(h, w2_ref[...], contract_t,
                                       preferred_element_type=jnp.float32
                                       ) + b2_ref[...]

    return pl.pallas_call(
        body,
        grid=(B // blk,),
        in_specs=[
            pl.BlockSpec((blk, D), lambda i: (i, 0)),
            pl.BlockSpec((H, D), lambda i: (0, 0)),
            pl.BlockSpec((1, H), lambda i: (0, 0)),
            pl.BlockSpec((O, H), lambda i: (0, 0)),
            pl.BlockSpec((1, O), lambda i: (0, 0)),
        ],
        out_specs=pl.BlockSpec((blk, O), lambda i: (i, 0)),
        out_shape=jax.ShapeDtypeStruct((B, O), jnp.float32),
    )(x, w1, b1.reshape(1, H), w2, b2.reshape(1, O))


def kernel(user_idx, table, W1, b1, W2, b2):
    x = _gather_rows_sc(table, user_idx.astype(jnp.int32))
    return _mlp_tc(x, W1, b1, W2, b2, blk=8192)


# speedup vs baseline: 2.4888x; 1.0028x over previous
import functools

import jax
import jax.numpy as jnp
from jax import lax
from jax.experimental import pallas as pl
from jax.experimental.pallas import tpu as pltpu
from jax.experimental.pallas import tpu_sc as plsc

_NUM_CORES = 2
_NUM_SUBCORES = 16
_NW = _NUM_CORES * _NUM_SUBCORES
_NCHUNK = 1


def _gather_rows_sc(table, idx):
    B = idx.shape[0]
    D = table.shape[1]
    b_per_w = B // _NW
    rows_per_chunk = b_per_w // _NCHUNK
    idx3 = idx.reshape(_NW, _NCHUNK, rows_per_chunk)

    mesh = plsc.VectorSubcoreMesh(
        core_axis_name="c", subcore_axis_name="s",
        num_cores=_NUM_CORES, num_subcores=_NUM_SUBCORES)

    @functools.partial(
        pl.kernel,
        mesh=mesh,
        compiler_params=pltpu.CompilerParams(
            skip_device_barrier=True,
            disable_bounds_checks=True,
            disable_semaphore_checks=True),
        out_type=jax.ShapeDtypeStruct((B, D), jnp.float32),
        scratch_types=[
            pltpu.VMEM((_NCHUNK, rows_per_chunk), jnp.int32),
            pltpu.VMEM((b_per_w, D), jnp.float32),
            pltpu.SemaphoreType.DMA,
            pltpu.SemaphoreType.DMA,
        ],
    )
    def gather_kernel(table_hbm, idx_hbm, out_hbm, idx_v, rows_v, gsem, wsem):
        wid = lax.axis_index("s") * _NUM_CORES + lax.axis_index("c")
        base = wid * b_per_w
        pltpu.sync_copy(idx_hbm.at[wid], idx_v)
        gathers = []
        for c in range(_NCHUNK):
            gathers.append(pltpu.async_copy(
                table_hbm.at[idx_v.at[c]],
                rows_v.at[pl.ds(c * rows_per_chunk, rows_per_chunk)],
                gsem))
        writes = []
        for c in range(_NCHUNK):
            gathers[c].wait()
            writes.append(pltpu.async_copy(
                rows_v.at[pl.ds(c * rows_per_chunk, rows_per_chunk)],
                out_hbm.at[pl.ds(base + c * rows_per_chunk, rows_per_chunk)],
                wsem))
        for w in writes:
            w.wait()

    return gather_kernel(table, idx3)


def _mlp_tc(x, w1, b1, w2, b2, blk):
    B, D = x.shape
    H, O = w1.shape[0], w2.shape[0]
    contract_t = (((1,), (1,)), ((), ()))

    def body(x_ref, w1_ref, b1_ref, w2_ref, b2_ref, out_ref):
        h = lax.dot_general(x_ref[...], w1_ref[...], contract_t,
                            preferred_element_type=jnp.float32)
        h = jnp.maximum(h + b1_ref[...], 0.0)
        out_ref[...] = lax.dot_general(h, w2_ref[...], contract_t,
                                       preferred_element_type=jnp.float32
                                       ) + b2_ref[...]

    return pl.pallas_call(
        body,
        grid=(B // blk,),
        in_specs=[
            pl.BlockSpec((blk, D), lambda i: (i, 0)),
            pl.BlockSpec((H, D), lambda i: (0, 0)),
            pl.BlockSpec((1, H), lambda i: (0, 0)),
            pl.BlockSpec((O, H), lambda i: (0, 0)),
            pl.BlockSpec((1, O), lambda i: (0, 0)),
        ],
        out_specs=pl.BlockSpec((blk, O), lambda i: (i, 0)),
        out_shape=jax.ShapeDtypeStruct((B, O), jnp.float32),
    )(x, w1, b1.reshape(1, H), w2, b2.reshape(1, O))


def kernel(user_idx, table, W1, b1, W2, b2):
    x = _gather_rows_sc(table, user_idx.astype(jnp.int32))
    return _mlp_tc(x, W1, b1, W2, b2, blk=8192)
